# baseline (speedup 1.0000x reference)
import functools

import jax
import jax.numpy as jnp
from jax.experimental import pallas as pl

_R16 = 16
_D = 32
_L = 5
_B = 1024
_PB = 8192


def _gru_segsum_kernel(pathT_ref, idx_row_ref, idx_col_ref, all16_ref,
                       edge_ref, W_ih_ref, W_hh_ref, b_ih_ref, b_hh_ref,
                       W_lin_ref, b_lin_ref, out_ref):
    f32 = jnp.float32
    dot = functools.partial(jax.lax.dot_general,
                            preferred_element_type=jnp.float32)
    mm_nt = (((1,), (1,)), ((), ()))
    W_ih = W_ih_ref[...]
    Ga = dot(W_ih, all16_ref[...], mm_nt)
    Ge = dot(W_ih, edge_ref[...], mm_nt)
    W_hh = W_hh_ref[...]
    b_ih = b_ih_ref[...]
    b_hh = b_hh_ref[...]

    mm = (((1,), (0,)), ((), ()))
    hT = jnp.zeros((_D, _PB), dtype=f32)
    iota16 = jax.lax.broadcasted_iota(jnp.int32, (_R16, _PB), 0)
    for t in range(_L):
        ids = pathT_ref[t, :]
        oh = (iota16 == ids[None, :]).astype(f32)
        G = Ga if t % 2 == 0 else Ge
        giT = dot(G, oh, mm) + b_ih
        ghT = dot(W_hh, hT, mm) + b_hh
        r = jax.nn.sigmoid(giT[0:_D] + ghT[0:_D])
        z = jax.nn.sigmoid(giT[_D:2 * _D] + ghT[_D:2 * _D])
        n = jnp.tanh(giT[2 * _D:] + r * ghT[2 * _D:])
        hT = (1.0 - z) * n + z * hT

    outT = dot(W_lin_ref[...], hT, mm) + b_lin_ref[...]

    idx_row = idx_row_ref[...]
    idx_col = idx_col_ref[...]
    iota8 = jax.lax.broadcasted_iota(jnp.int32, (8, _PB), 0)
    oh8w = jnp.where(iota8 == idx_row // 128, outT, 0.0)
    iota128 = jax.lax.broadcasted_iota(jnp.int32, (_PB, 128), 1)
    oh128 = (iota128 == idx_col % 128).astype(f32)
    contrib = dot(oh8w, oh128, mm)

    @pl.when(pl.program_id(0) == 0)
    def _init():
        out_ref[...] = jnp.zeros_like(out_ref)

    out_ref[...] += contrib


def kernel(users, path, path_idx, all_embedding, edge_embedding,
           virtual_embedding, W_ih, W_hh, b_ih, b_hh, W_lin, b_lin,
           interpret=False):
    del users, virtual_embedding
    P = path.shape[0]
    grid = (P // _PB,)
    pathT = path.T
    idx_row = path_idx.reshape(1, P)
    idx_col = path_idx.reshape(P, 1)
    const = lambda *_: (0, 0)
    score8 = pl.pallas_call(
        _gru_segsum_kernel,
        grid=grid,
        in_specs=[
            pl.BlockSpec((_L, _PB), lambda i: (0, i)),
            pl.BlockSpec((1, _PB), lambda i: (0, i)),
            pl.BlockSpec((_PB, 1), lambda i: (i, 0)),
            pl.BlockSpec((_R16, _D), const),
            pl.BlockSpec((_R16, _D), const),
            pl.BlockSpec((3 * _D, _D), const),
            pl.BlockSpec((3 * _D, _D), const),
            pl.BlockSpec((3 * _D, 1), const),
            pl.BlockSpec((3 * _D, 1), const),
            pl.BlockSpec((1, _D), const),
            pl.BlockSpec((1, 1), const),
        ],
        out_specs=pl.BlockSpec((8, 128), const),
        out_shape=jax.ShapeDtypeStruct((8, 128), jnp.float32),
        interpret=interpret,
    )(pathT, idx_row, idx_col, all_embedding, edge_embedding,
      W_ih, W_hh, b_ih.reshape(3 * _D, 1), b_hh.reshape(3 * _D, 1),
      W_lin, b_lin.reshape(1, 1))
    return score8.reshape(_B, 1)


# speedup vs baseline: 10.3473x; 10.3473x over previous
import functools

import jax
import jax.numpy as jnp
from jax.experimental import pallas as pl

_R16 = 16
_D = 32
_L = 5
_B = 1024
_PB = 8192


def _gru_segsum_kernel(pathT_ref, idx_row_ref, idx_col_ref, all16_ref,
                       edge_ref, W_ih_ref, W_hh_ref, b_ih_ref, b_hh_ref,
                       W_lin_ref, b_lin_ref, out_ref):
    f32 = jnp.float32
    dot = functools.partial(jax.lax.dot_general,
                            preferred_element_type=jnp.float32)
    mm_nt = (((1,), (1,)), ((), ()))
    W_ih = W_ih_ref[...]
    Ga = dot(W_ih, all16_ref[...], mm_nt)
    Ge = dot(W_ih, edge_ref[...], mm_nt)
    W_hh = W_hh_ref[...]
    b_ih = b_ih_ref[...]
    b_hh = b_hh_ref[...]

    mm = (((1,), (0,)), ((), ()))
    hT = jnp.zeros((_D, _PB), dtype=f32)
    iota16 = jax.lax.broadcasted_iota(jnp.int32, (_R16, _PB), 0)
    for t in range(_L):
        ids = pathT_ref[t, :]
        oh = (iota16 == ids[None, :]).astype(f32)
        G = Ga if t % 2 == 0 else Ge
        giT = dot(G, oh, mm) + b_ih
        ghT = dot(W_hh, hT, mm) + b_hh
        r = jax.nn.sigmoid(giT[0:_D] + ghT[0:_D])
        z = jax.nn.sigmoid(giT[_D:2 * _D] + ghT[_D:2 * _D])
        n = jnp.tanh(giT[2 * _D:] + r * ghT[2 * _D:])
        hT = (1.0 - z) * n + z * hT

    outT = dot(W_lin_ref[...], hT, mm) + b_lin_ref[...]

    idx_row = idx_row_ref[...]
    idx_col = idx_col_ref[...]
    iota8 = jax.lax.broadcasted_iota(jnp.int32, (8, _PB), 0)
    oh8w = jnp.where(iota8 == idx_row // 128, outT, 0.0)
    iota128 = jax.lax.broadcasted_iota(jnp.int32, (_PB, 128), 1)
    oh128 = (iota128 == idx_col % 128).astype(f32)
    contrib = dot(oh8w, oh128, mm)

    @pl.when(pl.program_id(0) == 0)
    def _init():
        out_ref[...] = jnp.zeros_like(out_ref)

    out_ref[...] += contrib


def kernel(users, path, path_idx, all_embedding, edge_embedding,
           virtual_embedding, W_ih, W_hh, b_ih, b_hh, W_lin, b_lin,
           interpret=False):
    del users, virtual_embedding
    P = path.shape[0]
    grid = (P // _PB,)
    all16 = jax.lax.slice(all_embedding, (0, 0), (_R16, _D))
    pathT = path.T
    idx_row = path_idx.reshape(1, P)
    idx_col = path_idx.reshape(P, 1)
    const = lambda *_: (0, 0)
    score8 = pl.pallas_call(
        _gru_segsum_kernel,
        grid=grid,
        in_specs=[
            pl.BlockSpec((_L, _PB), lambda i: (0, i)),
            pl.BlockSpec((1, _PB), lambda i: (0, i)),
            pl.BlockSpec((_PB, 1), lambda i: (i, 0)),
            pl.BlockSpec((_R16, _D), const),
            pl.BlockSpec((_R16, _D), const),
            pl.BlockSpec((3 * _D, _D), const),
            pl.BlockSpec((3 * _D, _D), const),
            pl.BlockSpec((3 * _D, 1), const),
            pl.BlockSpec((3 * _D, 1), const),
            pl.BlockSpec((1, _D), const),
            pl.BlockSpec((1, 1), const),
        ],
        out_specs=pl.BlockSpec((8, 128), const),
        out_shape=jax.ShapeDtypeStruct((8, 128), jnp.float32),
        interpret=interpret,
    )(pathT, idx_row, idx_col, all16, edge_embedding,
      W_ih, W_hh, b_ih.reshape(3 * _D, 1), b_hh.reshape(3 * _D, 1),
      W_lin, b_lin.reshape(1, 1))
    return score8.reshape(_B, 1)
